# Initial kernel scaffold; baseline (speedup 1.0000x reference)
import jax
import jax.numpy as jnp
from jax.experimental import pallas as pl


def kernel(x, vec, edge_index, edge_rbf, edge_vector, ln_gamma, ln_beta, W1, b1, W2, b2, Wrbf, brbf):
    raise NotImplementedError("write your pallas kernel here")


# speedup vs baseline: 14.6968x; 14.6968x over previous
import functools
import math

import jax
import jax.numpy as jnp
from jax import lax
from jax.experimental import pallas as pl
from jax.experimental.pallas import tpu as pltpu
from jax.experimental.pallas import tpu_sc as plsc

N = 10000
E = 160000
H = 128
H3 = 3 * H
NUM_RBF = 128
TBL = H3 + H3
MSG = 4 * H

_SILU_SCALE = 1.0 / 0.6
_INV_SQRT_3 = 1.0 / math.sqrt(3.0)
_INV_SQRT_H = 1.0 / math.sqrt(H)

NC = 2
NS = 16
NW = NC * NS

_BN = 400


def _node_mlp_body(x_ref, vec_ref, g_ref, b_ref, w1_ref, b1_ref, w2_ref,
                   b2_ref, out_ref):
    x = x_ref[...]
    mean = jnp.mean(x, axis=-1, keepdims=True)
    xc = x - mean
    var = jnp.mean(xc * xc, axis=-1, keepdims=True)
    xn = xc * lax.rsqrt(var + 1e-5) * g_ref[...] + b_ref[...]
    a = xn @ w1_ref[...] + b1_ref[...]
    a = jax.nn.silu(a) * _SILU_SCALE
    h3 = a @ w2_ref[...] + b2_ref[...]
    out_ref[:, :H3] = h3
    out_ref[:, H3:] = vec_ref[...]


def _node_mlp(x, vec2, g, b, w1, b1, w2, b2):
    return pl.pallas_call(
        _node_mlp_body,
        grid=(N // _BN,),
        in_specs=[
            pl.BlockSpec((_BN, H), lambda i: (i, 0)),
            pl.BlockSpec((_BN, H3), lambda i: (i, 0)),
            pl.BlockSpec((1, H), lambda i: (0, 0)),
            pl.BlockSpec((1, H), lambda i: (0, 0)),
            pl.BlockSpec((H, H), lambda i: (0, 0)),
            pl.BlockSpec((1, H), lambda i: (0, 0)),
            pl.BlockSpec((H, H3), lambda i: (0, 0)),
            pl.BlockSpec((1, H3), lambda i: (0, 0)),
        ],
        out_specs=pl.BlockSpec((_BN, TBL), lambda i: (i, 0)),
        out_shape=jax.ShapeDtypeStruct((N, TBL), jnp.float32),
    )(x, vec2, g, b, w1, b1, w2, b2)


_EPW = E // NW
_BG = 128
_NBG = _EPW // _BG
_TG = _EPW - _NBG * _BG

@functools.cache
def _get_gather_k():
    mesh = plsc.VectorSubcoreMesh(core_axis_name="c", subcore_axis_name="s")
    return functools.partial(
        pl.kernel,
        out_type=jax.ShapeDtypeStruct((E, TBL), jnp.float32),
        mesh=mesh,
        scratch_types=[
            pltpu.VMEM((_BG,), jnp.int32),
            pltpu.VMEM((_TG,), jnp.int32),
            pltpu.VMEM((_BG, TBL), jnp.float32),
            pltpu.SemaphoreType.DMA,
        ],
    )(_gather_body)


def _gather_body(table_hbm, src_hbm, out_hbm, idx_v, idxt_v, rows_v, sem):
    wid = lax.axis_index("s") * NC + lax.axis_index("c")
    base = wid * _EPW

    def body(bi, carry):
        off = base + bi * _BG
        pltpu.sync_copy(src_hbm.at[pl.ds(off, _BG)], idx_v)
        pltpu.async_copy(table_hbm.at[idx_v], rows_v, sem).wait()
        pltpu.sync_copy(rows_v, out_hbm.at[pl.ds(off, _BG)])
        return carry

    lax.fori_loop(0, _NBG, body, 0)
    toff = base + _NBG * _BG
    pltpu.sync_copy(src_hbm.at[pl.ds(toff, _TG)], idxt_v)
    pltpu.async_copy(table_hbm.at[idxt_v], rows_v.at[pl.ds(0, _TG)], sem).wait()
    pltpu.sync_copy(rows_v.at[pl.ds(0, _TG)], out_hbm.at[pl.ds(toff, _TG)])


_BE = 640


def _msg_body(rbf_ref, jv_ref, ev_ref, wr_ref, br_ref, out_ref):
    rbfh = rbf_ref[...] @ wr_ref[...] + br_ref[...]
    jv = jv_ref[...]
    prod = jv[:, :H3] * rbfh
    x_m = prod[:, :H]
    xh2 = prod[:, H:2 * H] * _INV_SQRT_3
    xh3 = prod[:, 2 * H:]
    out_ref[:, :H] = x_m
    for d in range(3):
        vec_d = jv[:, H3 + d * H:H3 + (d + 1) * H]
        ev_d = ev_ref[:, d:d + 1]
        out_ref[:, (d + 1) * H:(d + 2) * H] = (
            vec_d * xh2 + xh3 * ev_d) * _INV_SQRT_H


def _messages(edge_rbf, jv, evp, wr, br):
    return pl.pallas_call(
        _msg_body,
        grid=(E // _BE,),
        in_specs=[
            pl.BlockSpec((_BE, NUM_RBF), lambda i: (i, 0)),
            pl.BlockSpec((_BE, TBL), lambda i: (i, 0)),
            pl.BlockSpec((_BE, 8), lambda i: (i, 0)),
            pl.BlockSpec((NUM_RBF, H3), lambda i: (0, 0)),
            pl.BlockSpec((1, H3), lambda i: (0, 0)),
        ],
        out_specs=pl.BlockSpec((_BE, MSG), lambda i: (i, 0)),
        out_shape=jax.ShapeDtypeStruct((E, MSG), jnp.float32),
    )(edge_rbf, jv, evp, wr, br)


_EPS = E // NS
_BS = 80
_NBS = _EPS // _BS
_NCH = MSG // H
_FL = 10
_FR = N // _FL


@functools.cache
def _get_scatter_k():
    mesh = plsc.VectorSubcoreMesh(core_axis_name="c", subcore_axis_name="s")
    return functools.partial(
        pl.kernel,
        out_type=jax.ShapeDtypeStruct((_NCH, N, H), jnp.float32),
        mesh=mesh,
        scratch_types=[
            pltpu.VMEM((_NBS, _BS), jnp.int32),
            pltpu.VMEM((_BS, H), jnp.float32),
            pltpu.VMEM_SHARED((N, H), jnp.float32),
        ],
    )(_scatter_body)


def _scatter_body(m_hbm, dst_hbm, zeros_hbm, out_hbm, idx_v, msg_v, acc_sh):
    cid = lax.axis_index("c")
    sid = lax.axis_index("s")
    pltpu.sync_copy(dst_hbm.at[sid], idx_v)
    for r in range(_NCH // NC):
        ch = r * NC + cid

        @pl.when(sid < _FL)
        def _zero():
            pltpu.sync_copy(zeros_hbm, acc_sh.at[pl.ds(sid * _FR, _FR)])

        plsc.subcore_barrier()

        def body(bi, carry):
            pltpu.sync_copy(
                m_hbm.at[pl.ds(sid * _EPS + bi * _BS, _BS),
                         pl.ds(ch * H, H)],
                msg_v)
            pltpu.sync_copy(msg_v, acc_sh.at[idx_v.at[bi]], add=True)
            return carry

        lax.fori_loop(0, _NBS, body, 0)
        plsc.subcore_barrier()

        @pl.when(sid < _FL)
        def _flush():
            pltpu.sync_copy(acc_sh.at[pl.ds(sid * _FR, _FR)],
                            out_hbm.at[ch, pl.ds(sid * _FR, _FR)])

        plsc.subcore_barrier()


def kernel(x, vec, edge_index, edge_rbf, edge_vector, ln_gamma, ln_beta,
           W1, b1, W2, b2, Wrbf, brbf):
    src = edge_index[0].astype(jnp.int32)
    dst = edge_index[1].astype(jnp.int32)
    vec2 = vec.reshape(N, H3)

    table = _node_mlp(x, vec2,
                      ln_gamma.reshape(1, H), ln_beta.reshape(1, H),
                      W1, b1.reshape(1, H), W2, b2.reshape(1, H3))
    jv = _get_gather_k()(table, src)
    evp = jnp.pad(edge_vector, ((0, 0), (0, 5)))
    m = _messages(edge_rbf, jv, evp, Wrbf, brbf.reshape(1, H3))

    dst_r = dst.reshape(NS, _NBS, _BS)
    zeros = jnp.zeros((_FR, H), jnp.float32)
    out4 = _get_scatter_k()(m, dst_r, zeros)
    dx = out4[0]
    dvec = jnp.transpose(out4[1:], (1, 0, 2))
    return dx, dvec
